# Initial kernel scaffold; baseline (speedup 1.0000x reference)
import jax
import jax.numpy as jnp
from jax.experimental import pallas as pl


def kernel(x, edge_index, W1, b1, W2, b2):
    raise NotImplementedError("write your pallas kernel here")


# speedup vs baseline: 11.5128x; 11.5128x over previous
import functools

import jax
import jax.numpy as jnp
from jax import lax
from jax.experimental import pallas as pl
from jax.experimental.pallas import tpu as pltpu
from jax.experimental.pallas import tpu_sc as plsc

_N = 10000
_NP = 10240
_E = 320000
_NC = 2
_NS = 16
_NW = _NC * _NS
_EPT = _E // _NW
_CHUNK = 80
_NCHUNK = _EPT // _CHUNK
_RPT = _NP // _NS


def _make_propagate(D, gather_rows):
    mesh = plsc.VectorSubcoreMesh(core_axis_name="c", subcore_axis_name="s")

    @functools.partial(
        pl.kernel,
        mesh=mesh,
        compiler_params=pltpu.CompilerParams(use_tc_tiling_on_sc=False),
        out_type=jax.ShapeDtypeStruct((_NC, _NP, D), jnp.float32),
        scratch_types=[
            pltpu.VMEM((_CHUNK,), jnp.int32),
            pltpu.VMEM((_CHUNK,), jnp.int32),
            pltpu.VMEM((_CHUNK, D), jnp.float32),
            pltpu.VMEM_SHARED((_NP, D), jnp.float32),
            pltpu.SemaphoreType.DMA,
        ],
    )
    def propagate(u_hbm, row_hbm, col_hbm, out_hbm, idxr, idxc, rows, acc, sem):
        c = lax.axis_index("c")
        s = lax.axis_index("s")
        wid = c * _NS + s
        pltpu.sync_copy(u_hbm.at[pl.ds(s * _RPT, _RPT)],
                        acc.at[pl.ds(s * _RPT, _RPT)])
        if not gather_rows:
            pltpu.sync_copy(u_hbm.at[pl.ds(0, _CHUNK)], rows)
        plsc.subcore_barrier()

        ebase = wid * _EPT

        def body(i, carry):
            b = ebase + i * _CHUNK
            pltpu.sync_copy(col_hbm.at[pl.ds(b, _CHUNK)], idxc)
            if gather_rows:
                pltpu.sync_copy(row_hbm.at[pl.ds(b, _CHUNK)], idxr)
                pltpu.async_copy(u_hbm.at[idxr], rows, sem).wait()
            pltpu.sync_copy(rows, acc.at[idxc], add=True)
            return carry

        lax.fori_loop(0, _NCHUNK, body, 0)
        plsc.subcore_barrier()
        pltpu.sync_copy(acc.at[pl.ds(s * _RPT, _RPT)],
                        out_hbm.at[c, pl.ds(s * _RPT, _RPT)])

    return propagate


_ROWS_B = 1024
_GRID = _NP // _ROWS_B


def _tc_call(body, out_widths, *args):
    in_specs = []
    ops = []
    for arr, kind in args:
        ops.append(arr)
        if kind == "rows":
            in_specs.append(
                pl.BlockSpec((_ROWS_B,) + arr.shape[1:],
                             lambda i: (i,) + (0,) * (arr.ndim - 1)))
        else:
            in_specs.append(
                pl.BlockSpec(arr.shape, lambda i: (0,) * arr.ndim))
    out_specs = [pl.BlockSpec((_ROWS_B, w), lambda i: (i, 0))
                 for w in out_widths]
    out_shape = [jax.ShapeDtypeStruct((_NP, w), jnp.float32)
                 for w in out_widths]
    res = pl.pallas_call(
        body,
        grid=(_GRID,),
        in_specs=in_specs,
        out_specs=out_specs if len(out_widths) > 1 else out_specs[0],
        out_shape=out_shape if len(out_widths) > 1 else out_shape[0],
    )(*ops)
    return res


def _deg_body(d0_ref, d1_ref, x_ref, w1_ref, dinv_ref, u1_ref):
    deg = d0_ref[:] + d1_ref[:] - 1.0
    dinv = lax.rsqrt(deg)
    dinv_ref[:] = dinv
    u1_ref[:] = jnp.dot(x_ref[:], w1_ref[:],
                        preferred_element_type=jnp.float32) * dinv


def _mid_body(p0_ref, p1_ref, u_ref, dinv_ref, w_ref):
    dinv = dinv_ref[:]
    w_ref[:] = (p0_ref[:] + p1_ref[:] - u_ref[:]) * dinv * dinv


def _layer1_out_body(q0_ref, q1_ref, w_ref, dinv_ref, b1_ref, w2_ref,
                     u2_ref):
    dinv = dinv_ref[:]
    z = (q0_ref[:] + q1_ref[:] - w_ref[:]) * dinv
    h = jnp.maximum(z + b1_ref[:], 0.0)
    u2_ref[:] = jnp.dot(h, w2_ref[:],
                        preferred_element_type=jnp.float32) * dinv


def _final_body(q0_ref, q1_ref, w_ref, dinv_ref, b2_ref, out_ref):
    t = (q0_ref[:] + q1_ref[:] - w_ref[:]) * dinv_ref[:] + b2_ref[:]
    t = t[:, :40]
    m = jnp.max(t, axis=1, keepdims=True)
    e = t - m
    out_ref[:] = e - jnp.log(jnp.sum(jnp.exp(e), axis=1, keepdims=True))


def kernel(x, edge_index, W1, b1, W2, b2):
    row = edge_index[0]
    col = edge_index[1]
    xp = jnp.pad(x, ((0, _NP - _N), (0, 0)))
    ones16 = jnp.ones((_NP, 16), jnp.float32)
    W2p = jnp.pad(W2, ((0, 0), (0, 8)))
    b2p = jnp.pad(b2, (0, 8))

    degp = _make_propagate(16, False)(ones16, row, col)
    d0 = degp[0, :, :1]
    d1 = degp[1, :, :1]

    dinv, u1 = _tc_call(_deg_body, (1, 64),
                        (d0, "rows"), (d1, "rows"),
                        (xp, "rows"), (W1, "full"))

    prop64 = _make_propagate(64, True)
    prop48 = _make_propagate(48, True)

    p = prop64(u1, row, col)
    w1v = _tc_call(_mid_body, (64,),
                   (p[0], "rows"), (p[1], "rows"),
                   (u1, "rows"), (dinv, "rows"))
    q = prop64(w1v, row, col)
    u2 = _tc_call(_layer1_out_body, (48,),
                  (q[0], "rows"), (q[1], "rows"),
                  (w1v, "rows"), (dinv, "rows"),
                  (b1.reshape(1, 64), "full"), (W2p, "full"))

    p2 = prop48(u2, row, col)
    w2v = _tc_call(_mid_body, (48,),
                   (p2[0], "rows"), (p2[1], "rows"),
                   (u2, "rows"), (dinv, "rows"))
    q2 = prop48(w2v, row, col)
    out = _tc_call(_final_body, (40,),
                   (q2[0], "rows"), (q2[1], "rows"),
                   (w2v, "rows"), (dinv, "rows"),
                   (b2p.reshape(1, 48), "full"))
    return out[:_N]


# speedup vs baseline: 13.1485x; 1.1421x over previous
import functools

import jax
import jax.numpy as jnp
from jax import lax
from jax.experimental import pallas as pl
from jax.experimental.pallas import tpu as pltpu
from jax.experimental.pallas import tpu_sc as plsc

_N = 10000
_NP = 10240
_E = 320000
_NC = 2
_NS = 16
_NW = _NC * _NS
_CHUNK = 128
_EPAD = 327680
_EPT = _EPAD // _NW
_NCHUNK = _EPT // _CHUNK
_PADNODE = 10200
_RPT = _NP // _NS


def _make_propagate(D, gather_rows):
    mesh = plsc.VectorSubcoreMesh(core_axis_name="c", subcore_axis_name="s")

    @functools.partial(
        pl.kernel,
        mesh=mesh,
        compiler_params=pltpu.CompilerParams(use_tc_tiling_on_sc=False),
        out_type=jax.ShapeDtypeStruct((_NC, _NP, D), jnp.float32),
        scratch_types=[
            pltpu.VMEM((_NCHUNK, _CHUNK), jnp.int32),
            pltpu.VMEM((_NCHUNK, _CHUNK), jnp.int32),
            pltpu.VMEM((_CHUNK, D), jnp.float32),
            pltpu.VMEM((_CHUNK, D), jnp.float32),
            pltpu.VMEM_SHARED((_NP, D), jnp.float32),
            pltpu.SemaphoreType.DMA,
            pltpu.SemaphoreType.DMA,
        ],
    )
    def propagate(u_hbm, row_hbm, col_hbm, out_hbm, idxr, idxc,
                  rows_a, rows_b, acc, sga, sgb):
        c = lax.axis_index("c")
        s = lax.axis_index("s")
        wid = c * _NS + s
        cbase = wid * _NCHUNK
        pltpu.sync_copy(u_hbm.at[pl.ds(s * _RPT, _RPT)],
                        acc.at[pl.ds(s * _RPT, _RPT)])
        pltpu.sync_copy(col_hbm.at[pl.ds(cbase, _NCHUNK)], idxc)
        if gather_rows:
            pltpu.sync_copy(row_hbm.at[pl.ds(cbase, _NCHUNK)], idxr)
        else:
            pltpu.sync_copy(u_hbm.at[pl.ds(0, _CHUNK)], rows_a)
        plsc.subcore_barrier()

        if not gather_rows:
            def body0(i, carry):
                pltpu.sync_copy(rows_a, acc.at[idxc.at[i]], add=True)
                return carry
            lax.fori_loop(0, _NCHUNK, body0, 0)
        else:
            pltpu.async_copy(u_hbm.at[idxr.at[0]], rows_a, sga)
            pltpu.async_copy(u_hbm.at[idxr.at[1]], rows_b, sgb)

            def body(g, carry):
                pltpu.make_async_copy(u_hbm.at[idxr.at[0]], rows_a,
                                      sga).wait()
                pltpu.sync_copy(rows_a, acc.at[idxc.at[2 * g]], add=True)
                pltpu.async_copy(u_hbm.at[idxr.at[2 * g + 2]], rows_a, sga)
                pltpu.make_async_copy(u_hbm.at[idxr.at[1]], rows_b,
                                      sgb).wait()
                pltpu.sync_copy(rows_b, acc.at[idxc.at[2 * g + 1]], add=True)
                pltpu.async_copy(u_hbm.at[idxr.at[2 * g + 3]], rows_b, sgb)
                return carry

            lax.fori_loop(0, _NCHUNK // 2 - 1, body, 0)
            pltpu.make_async_copy(u_hbm.at[idxr.at[0]], rows_a, sga).wait()
            pltpu.sync_copy(rows_a, acc.at[idxc.at[_NCHUNK - 2]], add=True)
            pltpu.make_async_copy(u_hbm.at[idxr.at[1]], rows_b, sgb).wait()
            pltpu.sync_copy(rows_b, acc.at[idxc.at[_NCHUNK - 1]], add=True)

        plsc.subcore_barrier()
        pltpu.sync_copy(acc.at[pl.ds(s * _RPT, _RPT)],
                        out_hbm.at[c, pl.ds(s * _RPT, _RPT)])

    return propagate


_ROWS_B = 1024
_GRID = _NP // _ROWS_B


def _tc_call(body, out_widths, *args):
    in_specs = []
    ops = []
    for arr, kind in args:
        ops.append(arr)
        if kind == "rows":
            in_specs.append(
                pl.BlockSpec((_ROWS_B,) + arr.shape[1:],
                             lambda i: (i,) + (0,) * (arr.ndim - 1)))
        else:
            in_specs.append(
                pl.BlockSpec(arr.shape, lambda i: (0,) * arr.ndim))
    out_specs = [pl.BlockSpec((_ROWS_B, w), lambda i: (i, 0))
                 for w in out_widths]
    out_shape = [jax.ShapeDtypeStruct((_NP, w), jnp.float32)
                 for w in out_widths]
    res = pl.pallas_call(
        body,
        grid=(_GRID,),
        in_specs=in_specs,
        out_specs=out_specs if len(out_widths) > 1 else out_specs[0],
        out_shape=out_shape if len(out_widths) > 1 else out_shape[0],
    )(*ops)
    return res


def _deg_body(d0_ref, d1_ref, x_ref, w1_ref, dinv_ref, u1_ref):
    deg = d0_ref[:] + d1_ref[:] - 1.0
    dinv = lax.rsqrt(deg)
    dinv_ref[:] = dinv
    u1_ref[:] = jnp.dot(x_ref[:], w1_ref[:],
                        preferred_element_type=jnp.float32) * dinv


def _mid_body(p0_ref, p1_ref, u_ref, dinv_ref, w_ref):
    dinv = dinv_ref[:]
    w_ref[:] = (p0_ref[:] + p1_ref[:] - u_ref[:]) * dinv * dinv


def _layer1_out_body(q0_ref, q1_ref, w_ref, dinv_ref, b1_ref, w2_ref,
                     u2_ref):
    dinv = dinv_ref[:]
    z = (q0_ref[:] + q1_ref[:] - w_ref[:]) * dinv
    h = jnp.maximum(z + b1_ref[:], 0.0)
    u2_ref[:] = jnp.dot(h, w2_ref[:],
                        preferred_element_type=jnp.float32) * dinv


def _final_body(q0_ref, q1_ref, w_ref, dinv_ref, b2_ref, out_ref):
    t = (q0_ref[:] + q1_ref[:] - w_ref[:]) * dinv_ref[:] + b2_ref[:]
    t = t[:, :40]
    m = jnp.max(t, axis=1, keepdims=True)
    e = t - m
    out_ref[:] = e - jnp.log(jnp.sum(jnp.exp(e), axis=1, keepdims=True))


def kernel(x, edge_index, W1, b1, W2, b2):
    epad = jnp.full((_EPAD - _E,), _PADNODE, jnp.int32)
    row = jnp.concatenate([edge_index[0], epad]).reshape(-1, _CHUNK)
    col = jnp.concatenate([edge_index[1], epad]).reshape(-1, _CHUNK)
    xp = jnp.pad(x, ((0, _NP - _N), (0, 0)))
    ones16 = jnp.ones((_NP, 16), jnp.float32)
    W2p = jnp.pad(W2, ((0, 0), (0, 8)))
    b2p = jnp.pad(b2, (0, 8))

    degp = _make_propagate(16, False)(ones16, row, col)
    d0 = degp[0, :, :1]
    d1 = degp[1, :, :1]

    dinv, u1 = _tc_call(_deg_body, (1, 64),
                        (d0, "rows"), (d1, "rows"),
                        (xp, "rows"), (W1, "full"))

    prop64 = _make_propagate(64, True)
    prop48 = _make_propagate(48, True)

    p = prop64(u1, row, col)
    w1v = _tc_call(_mid_body, (64,),
                   (p[0], "rows"), (p[1], "rows"),
                   (u1, "rows"), (dinv, "rows"))
    q = prop64(w1v, row, col)
    u2 = _tc_call(_layer1_out_body, (48,),
                  (q[0], "rows"), (q[1], "rows"),
                  (w1v, "rows"), (dinv, "rows"),
                  (b1.reshape(1, 64), "full"), (W2p, "full"))

    p2 = prop48(u2, row, col)
    w2v = _tc_call(_mid_body, (48,),
                   (p2[0], "rows"), (p2[1], "rows"),
                   (u2, "rows"), (dinv, "rows"))
    q2 = prop48(w2v, row, col)
    out = _tc_call(_final_body, (40,),
                   (q2[0], "rows"), (q2[1], "rows"),
                   (w2v, "rows"), (dinv, "rows"),
                   (b2p.reshape(1, 48), "full"))
    return out[:_N]


# speedup vs baseline: 30.0594x; 2.2861x over previous
import functools

import jax
import jax.numpy as jnp
from jax import lax
from jax.experimental import pallas as pl
from jax.experimental.pallas import tpu as pltpu
from jax.experimental.pallas import tpu_sc as plsc

_N = 10000
_NP = 10240
_E = 320000
_NC = 2
_NS = 16
_NW = _NC * _NS
_CHUNK = 128
_EPAD = 327680
_EPT = _EPAD // _NW
_NCHUNK = _EPT // _CHUNK
_PADNODE = 10200
_RPT = _NP // _NS


def _make_propagate(D, gather_rows):
    mesh = plsc.VectorSubcoreMesh(core_axis_name="c", subcore_axis_name="s")

    @functools.partial(
        pl.kernel,
        mesh=mesh,
        compiler_params=pltpu.CompilerParams(use_tc_tiling_on_sc=False),
        out_type=jax.ShapeDtypeStruct((_NC, _NP, D), jnp.float32),
        scratch_types=[
            pltpu.VMEM((_NCHUNK, _CHUNK), jnp.int32),
            pltpu.VMEM((_NCHUNK, _CHUNK), jnp.int32),
            pltpu.VMEM((_CHUNK, D), jnp.float32),
            pltpu.VMEM((_CHUNK, D), jnp.float32),
            pltpu.VMEM_SHARED((_NP, D), jnp.float32),
            pltpu.SemaphoreType.DMA,
            pltpu.SemaphoreType.DMA,
        ],
    )
    def propagate(u_hbm, row_hbm, col_hbm, out_hbm, idxr, idxc,
                  rows_a, rows_b, acc, sga, sgb):
        c = lax.axis_index("c")
        s = lax.axis_index("s")
        wid = c * _NS + s
        cbase = wid * _NCHUNK
        pltpu.sync_copy(u_hbm.at[pl.ds(s * _RPT, _RPT)],
                        acc.at[pl.ds(s * _RPT, _RPT)])
        pltpu.sync_copy(col_hbm.at[pl.ds(cbase, _NCHUNK)], idxc)
        if gather_rows:
            pltpu.sync_copy(row_hbm.at[pl.ds(cbase, _NCHUNK)], idxr)
        else:
            pltpu.sync_copy(u_hbm.at[pl.ds(0, _CHUNK)], rows_a)
        plsc.subcore_barrier()

        if not gather_rows:
            def body0(i, carry):
                pltpu.sync_copy(rows_a, acc.at[idxc.at[i]], add=True)
                return carry
            lax.fori_loop(0, _NCHUNK, body0, 0)
        else:
            pltpu.async_copy(u_hbm.at[idxr.at[0]], rows_a, sga)
            pltpu.async_copy(u_hbm.at[idxr.at[1]], rows_b, sgb)

            def body(g, carry):
                pltpu.make_async_copy(u_hbm.at[idxr.at[0]], rows_a,
                                      sga).wait()
                pltpu.sync_copy(rows_a, acc.at[idxc.at[2 * g]], add=True)
                pltpu.async_copy(u_hbm.at[idxr.at[2 * g + 2]], rows_a, sga)
                pltpu.make_async_copy(u_hbm.at[idxr.at[1]], rows_b,
                                      sgb).wait()
                pltpu.sync_copy(rows_b, acc.at[idxc.at[2 * g + 1]], add=True)
                pltpu.async_copy(u_hbm.at[idxr.at[2 * g + 3]], rows_b, sgb)
                return carry

            lax.fori_loop(0, _NCHUNK // 2 - 1, body, 0)
            pltpu.make_async_copy(u_hbm.at[idxr.at[0]], rows_a, sga).wait()
            pltpu.sync_copy(rows_a, acc.at[idxc.at[_NCHUNK - 2]], add=True)
            pltpu.make_async_copy(u_hbm.at[idxr.at[1]], rows_b, sgb).wait()
            pltpu.sync_copy(rows_b, acc.at[idxc.at[_NCHUNK - 1]], add=True)

        plsc.subcore_barrier()
        pltpu.sync_copy(acc.at[pl.ds(s * _RPT, _RPT)],
                        out_hbm.at[c, pl.ds(s * _RPT, _RPT)])

    return propagate


_ROWS_B = 1024
_GRID = _NP // _ROWS_B


def _tc_call(body, out_widths, *args):
    in_specs = []
    ops = []
    for arr, kind in args:
        ops.append(arr)
        if kind == "rows":
            in_specs.append(
                pl.BlockSpec((_ROWS_B,) + arr.shape[1:],
                             lambda i: (i,) + (0,) * (arr.ndim - 1)))
        else:
            in_specs.append(
                pl.BlockSpec(arr.shape, lambda i: (0,) * arr.ndim))
    out_specs = [pl.BlockSpec((_ROWS_B, w), lambda i: (i, 0))
                 for w in out_widths]
    out_shape = [jax.ShapeDtypeStruct((_NP, w), jnp.float32)
                 for w in out_widths]
    res = pl.pallas_call(
        body,
        grid=(_GRID,),
        in_specs=in_specs,
        out_specs=out_specs if len(out_widths) > 1 else out_specs[0],
        out_shape=out_shape if len(out_widths) > 1 else out_shape[0],
    )(*ops)
    return res


def _deg_body(d0_ref, d1_ref, x_ref, w1_ref, dinv_ref, u1_ref):
    deg = d0_ref[:] + d1_ref[:] - 1.0
    dinv = lax.rsqrt(deg)
    dinv_ref[:] = dinv
    u1_ref[:] = jnp.dot(x_ref[:], w1_ref[:],
                        preferred_element_type=jnp.float32) * dinv


def _mid_body(p0_ref, p1_ref, u_ref, dinv_ref, w_ref):
    dinv = dinv_ref[:]
    w_ref[:] = (p0_ref[:] + p1_ref[:] - u_ref[:]) * dinv * dinv


def _layer1_out_body(q0_ref, q1_ref, w_ref, dinv_ref, b1_ref, w2_ref,
                     u2_ref):
    dinv = dinv_ref[:]
    z = (q0_ref[:] + q1_ref[:] - w_ref[:]) * dinv
    h = jnp.maximum(z + b1_ref[:], 0.0)
    u2_ref[:] = jnp.dot(h, w2_ref[:],
                        preferred_element_type=jnp.float32) * dinv


def _final_body(q0_ref, q1_ref, w_ref, dinv_ref, b2_ref, out_ref):
    t = (q0_ref[:] + q1_ref[:] - w_ref[:]) * dinv_ref[:] + b2_ref[:]
    t = t[:, :40]
    m = jnp.max(t, axis=1, keepdims=True)
    e = t - m
    out_ref[:] = e - jnp.log(jnp.sum(jnp.exp(e), axis=1, keepdims=True))


def kernel(x, edge_index, W1, b1, W2, b2):
    epad = _N + jnp.arange(_EPAD - _E, dtype=jnp.int32) % (_NP - _N)
    row = jnp.concatenate([edge_index[0], epad]).reshape(-1, _CHUNK)
    col = jnp.concatenate([edge_index[1], epad]).reshape(-1, _CHUNK)
    xp = jnp.pad(x, ((0, _NP - _N), (0, 0)))
    ones16 = jnp.ones((_NP, 16), jnp.float32)
    W2p = jnp.pad(W2, ((0, 0), (0, 8)))
    b2p = jnp.pad(b2, (0, 8))

    degp = _make_propagate(16, False)(ones16, row, col)
    d0 = degp[0, :, :1]
    d1 = degp[1, :, :1]

    dinv, u1 = _tc_call(_deg_body, (1, 64),
                        (d0, "rows"), (d1, "rows"),
                        (xp, "rows"), (W1, "full"))

    prop64 = _make_propagate(64, True)
    prop48 = _make_propagate(48, True)

    p = prop64(u1, row, col)
    w1v = _tc_call(_mid_body, (64,),
                   (p[0], "rows"), (p[1], "rows"),
                   (u1, "rows"), (dinv, "rows"))
    q = prop64(w1v, row, col)
    u2 = _tc_call(_layer1_out_body, (48,),
                  (q[0], "rows"), (q[1], "rows"),
                  (w1v, "rows"), (dinv, "rows"),
                  (b1.reshape(1, 64), "full"), (W2p, "full"))

    p2 = prop48(u2, row, col)
    w2v = _tc_call(_mid_body, (48,),
                   (p2[0], "rows"), (p2[1], "rows"),
                   (u2, "rows"), (dinv, "rows"))
    q2 = prop48(w2v, row, col)
    out = _tc_call(_final_body, (40,),
                   (q2[0], "rows"), (q2[1], "rows"),
                   (w2v, "rows"), (dinv, "rows"),
                   (b2p.reshape(1, 48), "full"))
    return out[:_N]


# speedup vs baseline: 30.1781x; 1.0039x over previous
import functools

import jax
import jax.numpy as jnp
from jax import lax
from jax.experimental import pallas as pl
from jax.experimental.pallas import tpu as pltpu
from jax.experimental.pallas import tpu_sc as plsc

_N = 10000
_NP = 10240
_E = 320000
_EPAD = 327680
_NC = 2
_NS = 16
_CHUNK = 128
_DH = 32
_RPT = _NP // _NS
_NCH = _EPAD // _NS // _CHUNK
_DEG_NCH = _EPAD // (_NC * _NS) // _CHUNK

_sc_mesh = plsc.VectorSubcoreMesh(core_axis_name="c", subcore_axis_name="s")
_sc_params = pltpu.CompilerParams(use_tc_tiling_on_sc=False)


@functools.partial(
    pl.kernel,
    mesh=_sc_mesh,
    compiler_params=_sc_params,
    out_type=jax.ShapeDtypeStruct((_NC, _NP, 16), jnp.float32),
    scratch_types=[
        pltpu.VMEM((_DEG_NCH, _CHUNK), jnp.int32),
        pltpu.VMEM((_CHUNK, 16), jnp.float32),
        pltpu.VMEM_SHARED((_NP, 16), jnp.float32),
    ],
)
def _degrees(ones_hbm, col_hbm, out_hbm, idxc, rows, acc):
    c = lax.axis_index("c")
    s = lax.axis_index("s")
    wid = c * _NS + s
    pltpu.sync_copy(ones_hbm.at[pl.ds(s * _RPT, _RPT)],
                    acc.at[pl.ds(s * _RPT, _RPT)])
    pltpu.sync_copy(col_hbm.at[pl.ds(wid * _DEG_NCH, _DEG_NCH)], idxc)
    pltpu.sync_copy(ones_hbm.at[pl.ds(0, _CHUNK)], rows)
    plsc.subcore_barrier()

    def body(i, carry):
        pltpu.sync_copy(rows, acc.at[idxc.at[i]], add=True)
        return carry

    lax.fori_loop(0, _DEG_NCH, body, 0)
    plsc.subcore_barrier()
    pltpu.sync_copy(acc.at[pl.ds(s * _RPT, _RPT)],
                    out_hbm.at[c, pl.ds(s * _RPT, _RPT)])


def _hop_pipeline(src, idxr, idxc, rows_a, rows_b, dst, sga, sgb):
    pltpu.async_copy(src.at[idxr.at[0]], rows_a, sga)
    pltpu.async_copy(src.at[idxr.at[1]], rows_b, sgb)

    def body(g, carry):
        pltpu.make_async_copy(src.at[idxr.at[0]], rows_a, sga).wait()
        pltpu.sync_copy(rows_a, dst.at[idxc.at[2 * g]], add=True)
        pltpu.async_copy(src.at[idxr.at[2 * g + 2]], rows_a, sga)
        pltpu.make_async_copy(src.at[idxr.at[1]], rows_b, sgb).wait()
        pltpu.sync_copy(rows_b, dst.at[idxc.at[2 * g + 1]], add=True)
        pltpu.async_copy(src.at[idxr.at[2 * g + 3]], rows_b, sgb)
        return carry

    lax.fori_loop(0, _NCH // 2 - 1, body, 0)
    pltpu.make_async_copy(src.at[idxr.at[0]], rows_a, sga).wait()
    pltpu.sync_copy(rows_a, dst.at[idxc.at[_NCH - 2]], add=True)
    pltpu.make_async_copy(src.at[idxr.at[1]], rows_b, sgb).wait()
    pltpu.sync_copy(rows_b, dst.at[idxc.at[_NCH - 1]], add=True)


@functools.partial(
    pl.kernel,
    mesh=_sc_mesh,
    compiler_params=_sc_params,
    out_type=jax.ShapeDtypeStruct((_NC, _NP, _DH), jnp.float32),
    scratch_types=[
        pltpu.VMEM((_NCH, _CHUNK), jnp.int32),
        pltpu.VMEM((_NCH, _CHUNK), jnp.int32),
        pltpu.VMEM((_CHUNK, _DH), jnp.float32),
        pltpu.VMEM((_CHUNK, _DH), jnp.float32),
        pltpu.VMEM((_RPT, _DH), jnp.float32),
        pltpu.VMEM((_RPT,), jnp.float32),
        pltpu.VMEM_SHARED((_NP, _DH), jnp.float32),
        pltpu.VMEM_SHARED((_NP, _DH), jnp.float32),
        pltpu.SemaphoreType.DMA,
        pltpu.SemaphoreType.DMA,
    ],
)
def _layer(u_hbm, dinv_hbm, row_hbm, col_hbm, out_hbm, idxr, idxc,
           rows_a, rows_b, stage, dv, acc1, acc2, sga, sgb):
    c = lax.axis_index("c")
    s = lax.axis_index("s")
    rb = s * _RPT
    pltpu.sync_copy(u_hbm.at[c, pl.ds(rb, _RPT)], acc1.at[pl.ds(rb, _RPT)])
    pltpu.sync_copy(row_hbm.at[pl.ds(s * _NCH, _NCH)], idxr)
    pltpu.sync_copy(col_hbm.at[pl.ds(s * _NCH, _NCH)], idxc)
    plsc.subcore_barrier()

    _hop_pipeline(u_hbm.at[c], idxr, idxc, rows_a, rows_b, acc1, sga, sgb)
    plsc.subcore_barrier()

    pltpu.sync_copy(dinv_hbm.at[pl.ds(rb, _RPT)], dv)
    pltpu.sync_copy(acc1.at[pl.ds(rb, _RPT)], stage)

    def scale(m, carry):
        dvec = dv[pl.ds(m * 16, 16)]
        for j in range(16):
            n = m * 16 + j
            d2 = dvec[j] * dvec[j]
            stage[n, pl.ds(0, 16)] = stage[n, pl.ds(0, 16)] * d2
            stage[n, pl.ds(16, 16)] = stage[n, pl.ds(16, 16)] * d2
        return carry

    lax.fori_loop(0, _RPT // 16, scale, 0)
    pltpu.sync_copy(stage, acc1.at[pl.ds(rb, _RPT)])
    pltpu.sync_copy(stage, acc2.at[pl.ds(rb, _RPT)])
    plsc.subcore_barrier()

    _hop_pipeline(acc1, idxr, idxc, rows_a, rows_b, acc2, sga, sgb)
    plsc.subcore_barrier()
    pltpu.sync_copy(acc2.at[pl.ds(rb, _RPT)], out_hbm.at[c, pl.ds(rb, _RPT)])


_ROWS_B = 1024
_GRID = _NP // _ROWS_B


def _entry_body(d0_ref, d1_ref, x_ref, w1_ref, dinv_ref, u_ref):
    deg = d0_ref[0, :, :1] + d1_ref[0, :, :1] - 1.0
    dinv = lax.rsqrt(deg)
    dinv_ref[:] = dinv
    u_ref[0] = jnp.dot(x_ref[:], w1_ref[0],
                       preferred_element_type=jnp.float32) * dinv


def _entry(degp, xp, W1):
    return pl.pallas_call(
        _entry_body,
        grid=(_NC, _GRID),
        in_specs=[
            pl.BlockSpec((1, _ROWS_B, 16), lambda c, i: (0, i, 0)),
            pl.BlockSpec((1, _ROWS_B, 16), lambda c, i: (1, i, 0)),
            pl.BlockSpec((_ROWS_B, 128), lambda c, i: (i, 0)),
            pl.BlockSpec((1, 128, _DH), lambda c, i: (c, 0, 0)),
        ],
        out_specs=[
            pl.BlockSpec((_ROWS_B, 1), lambda c, i: (i, 0)),
            pl.BlockSpec((1, _ROWS_B, _DH), lambda c, i: (c, i, 0)),
        ],
        out_shape=[
            jax.ShapeDtypeStruct((_NP, 1), jnp.float32),
            jax.ShapeDtypeStruct((_NC, _NP, _DH), jnp.float32),
        ],
    )(degp, degp, xp, W1)


def _mid_body(z0_ref, z1_ref, dinv_ref, b1_ref, w2_ref, u_ref):
    dinv = dinv_ref[:]
    z = jnp.concatenate([z0_ref[0], z1_ref[0]], axis=1)
    h = jnp.maximum(z * dinv + b1_ref[:], 0.0)
    u_ref[0] = jnp.dot(h, w2_ref[0],
                       preferred_element_type=jnp.float32) * dinv


def _mid(z3, dinv, b1, W2p):
    return pl.pallas_call(
        _mid_body,
        grid=(_NC, _GRID),
        in_specs=[
            pl.BlockSpec((1, _ROWS_B, _DH), lambda c, i: (0, i, 0)),
            pl.BlockSpec((1, _ROWS_B, _DH), lambda c, i: (1, i, 0)),
            pl.BlockSpec((_ROWS_B, 1), lambda c, i: (i, 0)),
            pl.BlockSpec((1, 2 * _DH), lambda c, i: (0, 0)),
            pl.BlockSpec((1, 2 * _DH, _DH), lambda c, i: (c, 0, 0)),
        ],
        out_specs=pl.BlockSpec((1, _ROWS_B, _DH), lambda c, i: (c, i, 0)),
        out_shape=jax.ShapeDtypeStruct((_NC, _NP, _DH), jnp.float32),
    )(z3, z3, dinv, b1, W2p)


def _final_body(z0_ref, z1_ref, dinv_ref, b2_ref, out_ref):
    z = jnp.concatenate([z0_ref[0], z1_ref[0]], axis=1)
    t = z * dinv_ref[:] + b2_ref[:]
    t = t[:, :40]
    m = jnp.max(t, axis=1, keepdims=True)
    e = t - m
    out_ref[:] = e - jnp.log(jnp.sum(jnp.exp(e), axis=1, keepdims=True))


def _final(z3, dinv, b2p):
    return pl.pallas_call(
        _final_body,
        grid=(_GRID,),
        in_specs=[
            pl.BlockSpec((1, _ROWS_B, _DH), lambda i: (0, i, 0)),
            pl.BlockSpec((1, _ROWS_B, _DH), lambda i: (1, i, 0)),
            pl.BlockSpec((_ROWS_B, 1), lambda i: (i, 0)),
            pl.BlockSpec((1, 2 * _DH), lambda i: (0, 0)),
        ],
        out_specs=pl.BlockSpec((_ROWS_B, 40), lambda i: (i, 0)),
        out_shape=jax.ShapeDtypeStruct((_NP, 40), jnp.float32),
    )(z3, z3, dinv, b2p)


def kernel(x, edge_index, W1, b1, W2, b2):
    epad = _N + jnp.arange(_EPAD - _E, dtype=jnp.int32) % (_NP - _N)
    row = jnp.concatenate([edge_index[0], epad]).reshape(-1, _CHUNK)
    col = jnp.concatenate([edge_index[1], epad]).reshape(-1, _CHUNK)
    xp = jnp.pad(x, ((0, _NP - _N), (0, 0)))
    ones16 = jnp.ones((_NP, 16), jnp.float32)
    W2f = jnp.pad(W2, ((0, 0), (0, 2 * _DH - 40)))
    W1c = jnp.stack([W1[:, :_DH], W1[:, _DH:]])
    W2p = jnp.stack([W2f[:, :_DH], W2f[:, _DH:]])
    b1r = b1.reshape(1, 2 * _DH)
    b2r = jnp.pad(b2, (0, 2 * _DH - 40)).reshape(1, 2 * _DH)

    degp = _degrees(ones16, col)
    dinv, u1 = _entry(degp, xp, W1c)
    dinv1 = dinv.reshape(_NP)
    z1 = _layer(u1, dinv1, row, col)
    u2 = _mid(z1, dinv, b1r, W2p)
    z2 = _layer(u2, dinv1, row, col)
    out = _final(z2, dinv, b2r)
    return out[:_N]


# speedup vs baseline: 35.9416x; 1.1910x over previous
import functools

import jax
import jax.numpy as jnp
from jax import lax
from jax.experimental import pallas as pl
from jax.experimental.pallas import tpu as pltpu
from jax.experimental.pallas import tpu_sc as plsc

_N = 10000
_NP = 10240
_E = 320000
_EPAD = 327680
_NC = 2
_NS = 16
_CHUNK = 128
_DH = 32
_RPT = _NP // _NS
_NCH = _EPAD // _NS // _CHUNK
_DEG_NCH = _EPAD // (_NC * _NS) // _CHUNK

_sc_mesh = plsc.VectorSubcoreMesh(core_axis_name="c", subcore_axis_name="s")
_sc_params = pltpu.CompilerParams(use_tc_tiling_on_sc=False,
                                  needs_layout_passes=False)


@functools.partial(
    pl.kernel,
    mesh=_sc_mesh,
    compiler_params=_sc_params,
    out_type=jax.ShapeDtypeStruct((_NC, _NP, 16), jnp.float32),
    scratch_types=[
        pltpu.VMEM((_DEG_NCH, _CHUNK), jnp.int32),
        pltpu.VMEM((_CHUNK, 16), jnp.float32),
        pltpu.VMEM_SHARED((_NP, 16), jnp.float32),
    ],
)
def _degrees(ones_hbm, col_hbm, out_hbm, idxc, rows, acc):
    c = lax.axis_index("c")
    s = lax.axis_index("s")
    wid = c * _NS + s
    pltpu.sync_copy(ones_hbm.at[pl.ds(s * _RPT, _RPT)],
                    acc.at[pl.ds(s * _RPT, _RPT)])
    pltpu.sync_copy(col_hbm.at[pl.ds(wid * _DEG_NCH, _DEG_NCH)], idxc)
    pltpu.sync_copy(ones_hbm.at[pl.ds(0, _CHUNK)], rows)
    plsc.subcore_barrier()

    def body(i, carry):
        pltpu.sync_copy(rows, acc.at[idxc.at[i]], add=True)
        return carry

    lax.fori_loop(0, _DEG_NCH, body, 0)
    plsc.subcore_barrier()
    pltpu.sync_copy(acc.at[pl.ds(s * _RPT, _RPT)],
                    out_hbm.at[c, pl.ds(s * _RPT, _RPT)])


def _hop_pipeline(src, idxr, idxc, rows_a, rows_b, dst, sga, sgb):
    pltpu.async_copy(src.at[idxr.at[0]], rows_a, sga)
    pltpu.async_copy(src.at[idxr.at[1]], rows_b, sgb)

    def body(g, carry):
        pltpu.make_async_copy(src.at[idxr.at[0]], rows_a, sga).wait()
        pltpu.sync_copy(rows_a, dst.at[idxc.at[2 * g]], add=True)
        pltpu.async_copy(src.at[idxr.at[2 * g + 2]], rows_a, sga)
        pltpu.make_async_copy(src.at[idxr.at[1]], rows_b, sgb).wait()
        pltpu.sync_copy(rows_b, dst.at[idxc.at[2 * g + 1]], add=True)
        pltpu.async_copy(src.at[idxr.at[2 * g + 3]], rows_b, sgb)
        return carry

    lax.fori_loop(0, _NCH // 2 - 1, body, 0)
    pltpu.make_async_copy(src.at[idxr.at[0]], rows_a, sga).wait()
    pltpu.sync_copy(rows_a, dst.at[idxc.at[_NCH - 2]], add=True)
    pltpu.make_async_copy(src.at[idxr.at[1]], rows_b, sgb).wait()
    pltpu.sync_copy(rows_b, dst.at[idxc.at[_NCH - 1]], add=True)


@functools.partial(
    pl.kernel,
    mesh=_sc_mesh,
    compiler_params=_sc_params,
    out_type=jax.ShapeDtypeStruct((_NC, _NP, _DH), jnp.bfloat16),
    scratch_types=[
        pltpu.VMEM((_NCH, _CHUNK), jnp.int32),
        pltpu.VMEM((_NCH, _CHUNK), jnp.int32),
        pltpu.VMEM((_CHUNK, _DH), jnp.bfloat16),
        pltpu.VMEM((_CHUNK, _DH), jnp.bfloat16),
        pltpu.VMEM((_RPT, _DH), jnp.bfloat16),
        pltpu.VMEM((_RPT,), jnp.float32),
        pltpu.VMEM_SHARED((_NP, _DH), jnp.bfloat16),
        pltpu.VMEM_SHARED((_NP, _DH), jnp.bfloat16),
        pltpu.SemaphoreType.DMA,
        pltpu.SemaphoreType.DMA,
    ],
)
def _layer(u_hbm, dinv_hbm, row_hbm, col_hbm, out_hbm, idxr, idxc,
           rows_a, rows_b, stage, dv, acc1, acc2, sga, sgb):
    c = lax.axis_index("c")
    s = lax.axis_index("s")
    rb = s * _RPT
    pltpu.sync_copy(u_hbm.at[c, pl.ds(rb, _RPT)], acc1.at[pl.ds(rb, _RPT)])
    pltpu.sync_copy(row_hbm.at[pl.ds(s * _NCH, _NCH)], idxr)
    pltpu.sync_copy(col_hbm.at[pl.ds(s * _NCH, _NCH)], idxc)
    plsc.subcore_barrier()

    _hop_pipeline(u_hbm.at[c], idxr, idxc, rows_a, rows_b, acc1, sga, sgb)
    plsc.subcore_barrier()

    pltpu.sync_copy(dinv_hbm.at[pl.ds(rb, _RPT)], dv)
    pltpu.sync_copy(acc1.at[pl.ds(rb, _RPT)], stage)

    def scale(m, carry):
        dvec = dv[pl.ds(m * 16, 16)]
        for j in range(16):
            n = m * 16 + j
            d2 = dvec[j] * dvec[j]
            va, vb = plsc.unpack(stage[n, :],
                                 format=plsc.PackFormat.INTERLEAVED)
            stage[n, :] = plsc.pack(va * d2, vb * d2,
                                    format=plsc.PackFormat.INTERLEAVED)
        return carry

    lax.fori_loop(0, _RPT // 16, scale, 0)
    pltpu.sync_copy(stage, acc1.at[pl.ds(rb, _RPT)])
    pltpu.sync_copy(stage, acc2.at[pl.ds(rb, _RPT)])
    plsc.subcore_barrier()

    _hop_pipeline(acc1, idxr, idxc, rows_a, rows_b, acc2, sga, sgb)
    plsc.subcore_barrier()
    pltpu.sync_copy(acc2.at[pl.ds(rb, _RPT)], out_hbm.at[c, pl.ds(rb, _RPT)])


_ROWS_B = 1024
_GRID = _NP // _ROWS_B


def _entry_body(d0_ref, d1_ref, x_ref, w1_ref, dinv_ref, u_ref):
    deg = d0_ref[0, :, :1] + d1_ref[0, :, :1] - 1.0
    dinv = lax.rsqrt(deg)
    dinv_ref[:] = dinv
    u_ref[0] = (jnp.dot(x_ref[:], w1_ref[0],
                        preferred_element_type=jnp.float32)
                * dinv).astype(jnp.bfloat16)


def _entry(degp, xp, W1):
    return pl.pallas_call(
        _entry_body,
        grid=(_NC, _GRID),
        in_specs=[
            pl.BlockSpec((1, _ROWS_B, 16), lambda c, i: (0, i, 0)),
            pl.BlockSpec((1, _ROWS_B, 16), lambda c, i: (1, i, 0)),
            pl.BlockSpec((_ROWS_B, 128), lambda c, i: (i, 0)),
            pl.BlockSpec((1, 128, _DH), lambda c, i: (c, 0, 0)),
        ],
        out_specs=[
            pl.BlockSpec((_ROWS_B, 1), lambda c, i: (i, 0)),
            pl.BlockSpec((1, _ROWS_B, _DH), lambda c, i: (c, i, 0)),
        ],
        out_shape=[
            jax.ShapeDtypeStruct((_NP, 1), jnp.float32),
            jax.ShapeDtypeStruct((_NC, _NP, _DH), jnp.bfloat16),
        ],
    )(degp, degp, xp, W1)


def _mid_body(z0_ref, z1_ref, dinv_ref, b1_ref, w2_ref, u_ref):
    dinv = dinv_ref[:]
    z = jnp.concatenate([z0_ref[0], z1_ref[0]],
                        axis=1).astype(jnp.float32)
    h = jnp.maximum(z * dinv + b1_ref[:], 0.0)
    u_ref[0] = (jnp.dot(h, w2_ref[0],
                        preferred_element_type=jnp.float32)
                * dinv).astype(jnp.bfloat16)


def _mid(z3, dinv, b1, W2p):
    return pl.pallas_call(
        _mid_body,
        grid=(_NC, _GRID),
        in_specs=[
            pl.BlockSpec((1, _ROWS_B, _DH), lambda c, i: (0, i, 0)),
            pl.BlockSpec((1, _ROWS_B, _DH), lambda c, i: (1, i, 0)),
            pl.BlockSpec((_ROWS_B, 1), lambda c, i: (i, 0)),
            pl.BlockSpec((1, 2 * _DH), lambda c, i: (0, 0)),
            pl.BlockSpec((1, 2 * _DH, _DH), lambda c, i: (c, 0, 0)),
        ],
        out_specs=pl.BlockSpec((1, _ROWS_B, _DH), lambda c, i: (c, i, 0)),
        out_shape=jax.ShapeDtypeStruct((_NC, _NP, _DH), jnp.bfloat16),
    )(z3, z3, dinv, b1, W2p)


def _final_body(z0_ref, z1_ref, dinv_ref, b2_ref, out_ref):
    z = jnp.concatenate([z0_ref[0], z1_ref[0]],
                        axis=1).astype(jnp.float32)
    t = z * dinv_ref[:] + b2_ref[:]
    t = t[:, :40]
    m = jnp.max(t, axis=1, keepdims=True)
    e = t - m
    out_ref[:] = e - jnp.log(jnp.sum(jnp.exp(e), axis=1, keepdims=True))


def _final(z3, dinv, b2p):
    return pl.pallas_call(
        _final_body,
        grid=(_GRID,),
        in_specs=[
            pl.BlockSpec((1, _ROWS_B, _DH), lambda i: (0, i, 0)),
            pl.BlockSpec((1, _ROWS_B, _DH), lambda i: (1, i, 0)),
            pl.BlockSpec((_ROWS_B, 1), lambda i: (i, 0)),
            pl.BlockSpec((1, 2 * _DH), lambda i: (0, 0)),
        ],
        out_specs=pl.BlockSpec((_ROWS_B, 40), lambda i: (i, 0)),
        out_shape=jax.ShapeDtypeStruct((_NP, 40), jnp.float32),
    )(z3, z3, dinv, b2p)


def kernel(x, edge_index, W1, b1, W2, b2):
    epad = _N + jnp.arange(_EPAD - _E, dtype=jnp.int32) % (_NP - _N)
    row = jnp.concatenate([edge_index[0], epad]).reshape(-1, _CHUNK)
    col = jnp.concatenate([edge_index[1], epad]).reshape(-1, _CHUNK)
    xp = jnp.pad(x, ((0, _NP - _N), (0, 0)))
    ones16 = jnp.ones((_NP, 16), jnp.float32)
    W2f = jnp.pad(W2, ((0, 0), (0, 2 * _DH - 40)))
    W1c = jnp.stack([W1[:, :_DH], W1[:, _DH:]])
    W2p = jnp.stack([W2f[:, :_DH], W2f[:, _DH:]])
    b1r = b1.reshape(1, 2 * _DH)
    b2r = jnp.pad(b2, (0, 2 * _DH - 40)).reshape(1, 2 * _DH)

    degp = _degrees(ones16, col)
    dinv, u1 = _entry(degp, xp, W1c)
    dinv1 = dinv.reshape(_NP)
    z1 = _layer(u1, dinv1, row, col)
    u2 = _mid(z1, dinv, b1r, W2p)
    z2 = _layer(u2, dinv1, row, col)
    out = _final(z2, dinv, b2r)
    return out[:_N]


# speedup vs baseline: 37.5244x; 1.0440x over previous
import functools

import jax
import jax.numpy as jnp
from jax import lax
from jax.experimental import pallas as pl
from jax.experimental.pallas import tpu as pltpu
from jax.experimental.pallas import tpu_sc as plsc

_N = 10000
_NP = 10240
_E = 320000
_EPAD = 327680
_NC = 2
_NS = 16
_CHUNK = 128
_DH = 32
_RPT = _NP // _NS
_NCH = _EPAD // _NS // _CHUNK
_DEG_NCH = _EPAD // (_NC * _NS) // _CHUNK

_sc_mesh = plsc.VectorSubcoreMesh(core_axis_name="c", subcore_axis_name="s")
_sc_params = pltpu.CompilerParams(use_tc_tiling_on_sc=False,
                                  needs_layout_passes=False)


@functools.partial(
    pl.kernel,
    mesh=_sc_mesh,
    compiler_params=_sc_params,
    out_type=jax.ShapeDtypeStruct((_NC, _NP, 16), jnp.float32),
    scratch_types=[
        pltpu.VMEM((_DEG_NCH, _CHUNK), jnp.int32),
        pltpu.VMEM((_CHUNK, 16), jnp.float32),
        pltpu.VMEM_SHARED((_NP, 16), jnp.float32),
    ],
)
def _degrees(ones_hbm, col_hbm, out_hbm, idxc, rows, acc):
    c = lax.axis_index("c")
    s = lax.axis_index("s")
    wid = c * _NS + s
    pltpu.sync_copy(ones_hbm.at[pl.ds(s * _RPT, _RPT)],
                    acc.at[pl.ds(s * _RPT, _RPT)])
    pltpu.sync_copy(col_hbm.at[pl.ds(wid * _DEG_NCH, _DEG_NCH)], idxc)
    pltpu.sync_copy(ones_hbm.at[pl.ds(0, _CHUNK)], rows)
    plsc.subcore_barrier()

    def body(i, carry):
        pltpu.sync_copy(rows, acc.at[idxc.at[i]], add=True)
        return carry

    lax.fori_loop(0, _DEG_NCH, body, 0)
    plsc.subcore_barrier()
    pltpu.sync_copy(acc.at[pl.ds(s * _RPT, _RPT)],
                    out_hbm.at[c, pl.ds(s * _RPT, _RPT)])


def _hop_pipeline(src, idxr, idxc, rows_a, rows_b, dst, sga, sgb):
    pltpu.async_copy(src.at[idxr.at[0]], rows_a, sga)
    pltpu.async_copy(src.at[idxr.at[1]], rows_b, sgb)

    def body(g, carry):
        pltpu.make_async_copy(src.at[idxr.at[0]], rows_a, sga).wait()
        pltpu.sync_copy(rows_a, dst.at[idxc.at[2 * g]], add=True)
        pltpu.async_copy(src.at[idxr.at[2 * g + 2]], rows_a, sga)
        pltpu.make_async_copy(src.at[idxr.at[1]], rows_b, sgb).wait()
        pltpu.sync_copy(rows_b, dst.at[idxc.at[2 * g + 1]], add=True)
        pltpu.async_copy(src.at[idxr.at[2 * g + 3]], rows_b, sgb)
        return carry

    lax.fori_loop(0, _NCH // 2 - 1, body, 0)
    pltpu.make_async_copy(src.at[idxr.at[0]], rows_a, sga).wait()
    pltpu.sync_copy(rows_a, dst.at[idxc.at[_NCH - 2]], add=True)
    pltpu.make_async_copy(src.at[idxr.at[1]], rows_b, sgb).wait()
    pltpu.sync_copy(rows_b, dst.at[idxc.at[_NCH - 1]], add=True)


@functools.partial(
    pl.kernel,
    mesh=_sc_mesh,
    compiler_params=_sc_params,
    out_type=jax.ShapeDtypeStruct((_NC, _NP, _DH), jnp.bfloat16),
    scratch_types=[
        pltpu.VMEM((_NCH, _CHUNK), jnp.int32),
        pltpu.VMEM((_NCH, _CHUNK), jnp.int32),
        pltpu.VMEM((_CHUNK, _DH), jnp.bfloat16),
        pltpu.VMEM((_CHUNK, _DH), jnp.bfloat16),
        pltpu.VMEM((_RPT, _DH), jnp.bfloat16),
        pltpu.VMEM((_RPT,), jnp.float32),
        pltpu.VMEM_SHARED((_NP, _DH), jnp.bfloat16),
        pltpu.VMEM_SHARED((_NP, _DH), jnp.bfloat16),
        pltpu.SemaphoreType.DMA,
        pltpu.SemaphoreType.DMA,
    ],
)
def _layer(u_hbm, dinv_hbm, row_hbm, col_hbm, out_hbm, idxr, idxc,
           rows_a, rows_b, stage, dv, acc1, acc2, sga, sgb):
    c = lax.axis_index("c")
    s = lax.axis_index("s")
    rb = s * _RPT
    pltpu.sync_copy(u_hbm.at[c, pl.ds(rb, _RPT)], acc1.at[pl.ds(rb, _RPT)])
    pltpu.sync_copy(row_hbm.at[pl.ds(s * _NCH, _NCH)], idxr)
    pltpu.sync_copy(col_hbm.at[pl.ds(s * _NCH, _NCH)], idxc)
    plsc.subcore_barrier()

    _hop_pipeline(u_hbm.at[c], idxr, idxc, rows_a, rows_b, acc1, sga, sgb)
    plsc.subcore_barrier()

    pltpu.sync_copy(dinv_hbm.at[pl.ds(rb, _RPT)], dv)
    pltpu.sync_copy(acc1.at[pl.ds(rb, _RPT)], stage)

    def scale(m, carry):
        dvec = dv[pl.ds(m * 16, 16)]
        for j in range(16):
            n = m * 16 + j
            d2 = dvec[j] * dvec[j]
            va, vb = plsc.unpack(stage[n, :],
                                 format=plsc.PackFormat.INTERLEAVED)
            stage[n, :] = plsc.pack(va * d2, vb * d2,
                                    format=plsc.PackFormat.INTERLEAVED)
        return carry

    lax.fori_loop(0, _RPT // 16, scale, 0)
    pltpu.sync_copy(stage, acc1.at[pl.ds(rb, _RPT)])
    pltpu.sync_copy(stage, acc2.at[pl.ds(rb, _RPT)])
    plsc.subcore_barrier()

    _hop_pipeline(acc1, idxr, idxc, rows_a, rows_b, acc2, sga, sgb)
    plsc.subcore_barrier()
    pltpu.sync_copy(acc2.at[pl.ds(rb, _RPT)], out_hbm.at[c, pl.ds(rb, _RPT)])


_ROWS_B = 1024
_GRID = _NP // _ROWS_B


def _entry_body(d0_ref, d1_ref, x_ref, w1_ref, dinv_ref, u_ref):
    deg = d0_ref[0, :, :1] + d1_ref[0, :, :1] - 1.0
    dinv = lax.rsqrt(deg)
    dinv_ref[:] = dinv
    for c in range(_NC):
        u_ref[c] = (jnp.dot(x_ref[:], w1_ref[c],
                            preferred_element_type=jnp.float32)
                    * dinv).astype(jnp.bfloat16)


def _entry(degp, xp, W1):
    return pl.pallas_call(
        _entry_body,
        grid=(_GRID,),
        in_specs=[
            pl.BlockSpec((1, _ROWS_B, 16), lambda i: (0, i, 0)),
            pl.BlockSpec((1, _ROWS_B, 16), lambda i: (1, i, 0)),
            pl.BlockSpec((_ROWS_B, 128), lambda i: (i, 0)),
            pl.BlockSpec((_NC, 128, _DH), lambda i: (0, 0, 0)),
        ],
        out_specs=[
            pl.BlockSpec((_ROWS_B, 1), lambda i: (i, 0)),
            pl.BlockSpec((_NC, _ROWS_B, _DH), lambda i: (0, i, 0)),
        ],
        out_shape=[
            jax.ShapeDtypeStruct((_NP, 1), jnp.float32),
            jax.ShapeDtypeStruct((_NC, _NP, _DH), jnp.bfloat16),
        ],
    )(degp, degp, xp, W1)


def _mid_body(z_ref, dinv_ref, b1_ref, w2_ref, u_ref):
    dinv = dinv_ref[:]
    z = jnp.concatenate([z_ref[0], z_ref[1]],
                        axis=1).astype(jnp.float32)
    h = jnp.maximum(z * dinv + b1_ref[:], 0.0)
    for c in range(_NC):
        u_ref[c] = (jnp.dot(h, w2_ref[c],
                            preferred_element_type=jnp.float32)
                    * dinv).astype(jnp.bfloat16)


def _mid(z3, dinv, b1, W2p):
    return pl.pallas_call(
        _mid_body,
        grid=(_GRID,),
        in_specs=[
            pl.BlockSpec((_NC, _ROWS_B, _DH), lambda i: (0, i, 0)),
            pl.BlockSpec((_ROWS_B, 1), lambda i: (i, 0)),
            pl.BlockSpec((1, 2 * _DH), lambda i: (0, 0)),
            pl.BlockSpec((_NC, 2 * _DH, _DH), lambda i: (0, 0, 0)),
        ],
        out_specs=pl.BlockSpec((_NC, _ROWS_B, _DH), lambda i: (0, i, 0)),
        out_shape=jax.ShapeDtypeStruct((_NC, _NP, _DH), jnp.bfloat16),
    )(z3, dinv, b1, W2p)


def _final_body(z_ref, dinv_ref, b2_ref, out_ref):
    z = jnp.concatenate([z_ref[0], z_ref[1]],
                        axis=1).astype(jnp.float32)
    t = z * dinv_ref[:] + b2_ref[:]
    t = t[:, :40]
    m = jnp.max(t, axis=1, keepdims=True)
    e = t - m
    out_ref[:] = e - jnp.log(jnp.sum(jnp.exp(e), axis=1, keepdims=True))


def _final(z3, dinv, b2p):
    return pl.pallas_call(
        _final_body,
        grid=(_GRID,),
        in_specs=[
            pl.BlockSpec((_NC, _ROWS_B, _DH), lambda i: (0, i, 0)),
            pl.BlockSpec((_ROWS_B, 1), lambda i: (i, 0)),
            pl.BlockSpec((1, 2 * _DH), lambda i: (0, 0)),
        ],
        out_specs=pl.BlockSpec((_ROWS_B, 40), lambda i: (i, 0)),
        out_shape=jax.ShapeDtypeStruct((_NP, 40), jnp.float32),
    )(z3, dinv, b2p)


def kernel(x, edge_index, W1, b1, W2, b2):
    epad = _N + jnp.arange(_EPAD - _E, dtype=jnp.int32) % (_NP - _N)
    row = jnp.concatenate([edge_index[0], epad]).reshape(-1, _CHUNK)
    col = jnp.concatenate([edge_index[1], epad]).reshape(-1, _CHUNK)
    xp = jnp.pad(x, ((0, _NP - _N), (0, 0)))
    ones16 = jnp.ones((_NP, 16), jnp.float32)
    W2f = jnp.pad(W2, ((0, 0), (0, 2 * _DH - 40)))
    W1c = jnp.stack([W1[:, :_DH], W1[:, _DH:]])
    W2p = jnp.stack([W2f[:, :_DH], W2f[:, _DH:]])
    b1r = b1.reshape(1, 2 * _DH)
    b2r = jnp.pad(b2, (0, 2 * _DH - 40)).reshape(1, 2 * _DH)

    degp = _degrees(ones16, col)
    dinv, u1 = _entry(degp, xp, W1c)
    dinv1 = dinv.reshape(_NP)
    z1 = _layer(u1, dinv1, row, col)
    u2 = _mid(z1, dinv, b1r, W2p)
    z2 = _layer(u2, dinv1, row, col)
    out = _final(z2, dinv, b2r)
    return out[:_N]


# speedup vs baseline: 38.0394x; 1.0137x over previous
import functools

import jax
import jax.numpy as jnp
from jax import lax
from jax.experimental import pallas as pl
from jax.experimental.pallas import tpu as pltpu
from jax.experimental.pallas import tpu_sc as plsc

_N = 10000
_NP = 10240
_E = 320000
_EPAD = 327680
_NC = 2
_NS = 16
_CHUNK = 128
_DH = 32
_RPT = _NP // _NS
_NCH = _EPAD // _NS // _CHUNK
_DEG_NCH = _EPAD // (_NC * _NS) // _CHUNK

_sc_mesh = plsc.VectorSubcoreMesh(core_axis_name="c", subcore_axis_name="s")
_sc_params = pltpu.CompilerParams(use_tc_tiling_on_sc=False,
                                  needs_layout_passes=False)


@functools.partial(
    pl.kernel,
    mesh=_sc_mesh,
    compiler_params=_sc_params,
    out_type=jax.ShapeDtypeStruct((_NC, _NP, 16), jnp.float32),
    scratch_types=[
        pltpu.VMEM((_DEG_NCH, _CHUNK), jnp.int32),
        pltpu.VMEM((_CHUNK, 16), jnp.float32),
        pltpu.VMEM_SHARED((_NP, 16), jnp.float32),
    ],
)
def _degrees(ones_hbm, col_hbm, out_hbm, idxc, rows, acc):
    c = lax.axis_index("c")
    s = lax.axis_index("s")
    wid = c * _NS + s
    pltpu.sync_copy(ones_hbm.at[pl.ds(s * _RPT, _RPT)],
                    acc.at[pl.ds(s * _RPT, _RPT)])
    pltpu.sync_copy(col_hbm.at[pl.ds(wid * _DEG_NCH, _DEG_NCH)], idxc)
    pltpu.sync_copy(ones_hbm.at[pl.ds(0, _CHUNK)], rows)
    plsc.subcore_barrier()

    def body(i, carry):
        pltpu.sync_copy(rows, acc.at[idxc.at[i]], add=True)
        return carry

    lax.fori_loop(0, _DEG_NCH, body, 0)
    plsc.subcore_barrier()
    pltpu.sync_copy(acc.at[pl.ds(s * _RPT, _RPT)],
                    out_hbm.at[c, pl.ds(s * _RPT, _RPT)])


def _hop_pipeline(src, idxr, idxc, rows_a, rows_b, dst, sga, sgb):
    pltpu.async_copy(src.at[idxr.at[0]], rows_a, sga)
    pltpu.async_copy(src.at[idxr.at[1]], rows_b, sgb)

    def body(g, carry):
        pltpu.make_async_copy(src.at[idxr.at[0]], rows_a, sga).wait()
        pltpu.sync_copy(rows_a, dst.at[idxc.at[2 * g]], add=True)
        pltpu.async_copy(src.at[idxr.at[2 * g + 2]], rows_a, sga)
        pltpu.make_async_copy(src.at[idxr.at[1]], rows_b, sgb).wait()
        pltpu.sync_copy(rows_b, dst.at[idxc.at[2 * g + 1]], add=True)
        pltpu.async_copy(src.at[idxr.at[2 * g + 3]], rows_b, sgb)
        return carry

    lax.fori_loop(0, _NCH // 2 - 1, body, 0)
    pltpu.make_async_copy(src.at[idxr.at[0]], rows_a, sga).wait()
    pltpu.sync_copy(rows_a, dst.at[idxc.at[_NCH - 2]], add=True)
    pltpu.make_async_copy(src.at[idxr.at[1]], rows_b, sgb).wait()
    pltpu.sync_copy(rows_b, dst.at[idxc.at[_NCH - 1]], add=True)


@functools.partial(
    pl.kernel,
    mesh=_sc_mesh,
    compiler_params=_sc_params,
    out_type=jax.ShapeDtypeStruct((_NC, _NP, _DH), jnp.bfloat16),
    scratch_types=[
        pltpu.VMEM((_NCH, _CHUNK), jnp.int32),
        pltpu.VMEM((_NCH, _CHUNK), jnp.int32),
        pltpu.VMEM((_CHUNK, _DH), jnp.bfloat16),
        pltpu.VMEM((_CHUNK, _DH), jnp.bfloat16),
        pltpu.VMEM((_RPT, _DH), jnp.bfloat16),
        pltpu.VMEM((_RPT,), jnp.float32),
        pltpu.VMEM_SHARED((_NP, _DH), jnp.bfloat16),
        pltpu.VMEM_SHARED((_NP, _DH), jnp.bfloat16),
        pltpu.SemaphoreType.DMA,
        pltpu.SemaphoreType.DMA,
    ],
)
def _layer(u_hbm, dinv_hbm, row_hbm, col_hbm, out_hbm, idxr, idxc,
           rows_a, rows_b, stage, dv, acc1, acc2, sga, sgb):
    c = lax.axis_index("c")
    s = lax.axis_index("s")
    rb = s * _RPT
    pltpu.sync_copy(u_hbm.at[c, pl.ds(rb, _RPT)], acc1.at[pl.ds(rb, _RPT)])
    pltpu.sync_copy(row_hbm.at[pl.ds(s * _NCH, _NCH)], idxr)
    pltpu.sync_copy(col_hbm.at[pl.ds(s * _NCH, _NCH)], idxc)
    plsc.subcore_barrier()

    _hop_pipeline(u_hbm.at[c], idxr, idxc, rows_a, rows_b, acc1, sga, sgb)
    plsc.subcore_barrier()

    pltpu.sync_copy(dinv_hbm.at[pl.ds(rb, _RPT)], dv)
    pltpu.sync_copy(acc1.at[pl.ds(rb, _RPT)], stage)

    def scale(m, carry):
        dvec = dv[pl.ds(m * 16, 16)]
        for j in range(16):
            n = m * 16 + j
            d2 = dvec[j] * dvec[j]
            va, vb = plsc.unpack(stage[n, :],
                                 format=plsc.PackFormat.INTERLEAVED)
            stage[n, :] = plsc.pack(va * d2, vb * d2,
                                    format=plsc.PackFormat.INTERLEAVED)
        return carry

    lax.fori_loop(0, _RPT // 16, scale, 0)
    pltpu.sync_copy(stage, acc1.at[pl.ds(rb, _RPT)])
    pltpu.sync_copy(stage, acc2.at[pl.ds(rb, _RPT)])
    plsc.subcore_barrier()

    _hop_pipeline(acc1, idxr, idxc, rows_a, rows_b, acc2, sga, sgb)
    plsc.subcore_barrier()
    pltpu.sync_copy(acc2.at[pl.ds(rb, _RPT)], out_hbm.at[c, pl.ds(rb, _RPT)])


_ROWS_B = 1024
_GRID = _NP // _ROWS_B


def _entry_body(d0_ref, d1_ref, x_ref, w1_ref, dinv_ref, dinvf_ref, u_ref):
    deg = d0_ref[0, :, :1] + d1_ref[0, :, :1] - 1.0
    dinv = lax.rsqrt(deg)
    dinv_ref[:] = dinv
    dinvf_ref[:] = dinv.reshape(_ROWS_B // 128, 128)
    for c in range(_NC):
        u_ref[c] = (jnp.dot(x_ref[:], w1_ref[c],
                            preferred_element_type=jnp.float32)
                    * dinv).astype(jnp.bfloat16)


def _entry(degp, xp, W1):
    return pl.pallas_call(
        _entry_body,
        grid=(_GRID,),
        in_specs=[
            pl.BlockSpec((1, _ROWS_B, 16), lambda i: (0, i, 0)),
            pl.BlockSpec((1, _ROWS_B, 16), lambda i: (1, i, 0)),
            pl.BlockSpec((_ROWS_B, 128), lambda i: (i, 0)),
            pl.BlockSpec((_NC, 128, _DH), lambda i: (0, 0, 0)),
        ],
        out_specs=[
            pl.BlockSpec((_ROWS_B, 1), lambda i: (i, 0)),
            pl.BlockSpec((_ROWS_B // 128, 128), lambda i: (i, 0)),
            pl.BlockSpec((_NC, _ROWS_B, _DH), lambda i: (0, i, 0)),
        ],
        out_shape=[
            jax.ShapeDtypeStruct((_NP, 1), jnp.float32),
            jax.ShapeDtypeStruct((_NP // 128, 128), jnp.float32),
            jax.ShapeDtypeStruct((_NC, _NP, _DH), jnp.bfloat16),
        ],
    )(degp, degp, xp, W1)


def _mid_body(z_ref, dinv_ref, b1_ref, w2_ref, u_ref):
    dinv = dinv_ref[:]
    z = jnp.concatenate([z_ref[0], z_ref[1]],
                        axis=1).astype(jnp.float32)
    h = jnp.maximum(z * dinv + b1_ref[:], 0.0)
    for c in range(_NC):
        u_ref[c] = (jnp.dot(h, w2_ref[c],
                            preferred_element_type=jnp.float32)
                    * dinv).astype(jnp.bfloat16)


def _mid(z3, dinv, b1, W2p):
    return pl.pallas_call(
        _mid_body,
        grid=(_GRID,),
        in_specs=[
            pl.BlockSpec((_NC, _ROWS_B, _DH), lambda i: (0, i, 0)),
            pl.BlockSpec((_ROWS_B, 1), lambda i: (i, 0)),
            pl.BlockSpec((1, 2 * _DH), lambda i: (0, 0)),
            pl.BlockSpec((_NC, 2 * _DH, _DH), lambda i: (0, 0, 0)),
        ],
        out_specs=pl.BlockSpec((_NC, _ROWS_B, _DH), lambda i: (0, i, 0)),
        out_shape=jax.ShapeDtypeStruct((_NC, _NP, _DH), jnp.bfloat16),
    )(z3, dinv, b1, W2p)


def _final_body(z_ref, dinv_ref, b2_ref, out_ref):
    z = jnp.concatenate([z_ref[0], z_ref[1]],
                        axis=1).astype(jnp.float32)
    t = z * dinv_ref[:] + b2_ref[:]
    t = t[:, :40]
    m = jnp.max(t, axis=1, keepdims=True)
    e = t - m
    out_ref[:] = e - jnp.log(jnp.sum(jnp.exp(e), axis=1, keepdims=True))


def _final(z3, dinv, b2p):
    return pl.pallas_call(
        _final_body,
        grid=(_GRID,),
        in_specs=[
            pl.BlockSpec((_NC, _ROWS_B, _DH), lambda i: (0, i, 0)),
            pl.BlockSpec((_ROWS_B, 1), lambda i: (i, 0)),
            pl.BlockSpec((1, 2 * _DH), lambda i: (0, 0)),
        ],
        out_specs=pl.BlockSpec((_ROWS_B, 40), lambda i: (i, 0)),
        out_shape=jax.ShapeDtypeStruct((_NP, 40), jnp.float32),
    )(z3, dinv, b2p)


def kernel(x, edge_index, W1, b1, W2, b2):
    epad = _N + jnp.arange(_EPAD - _E, dtype=jnp.int32) % (_NP - _N)
    row = jnp.concatenate([edge_index[0], epad]).reshape(-1, _CHUNK)
    col = jnp.concatenate([edge_index[1], epad]).reshape(-1, _CHUNK)
    xp = jnp.pad(x, ((0, _NP - _N), (0, 0)))
    ones16 = jnp.ones((_NP, 16), jnp.float32)
    W2f = jnp.pad(W2, ((0, 0), (0, 2 * _DH - 40)))
    W1c = jnp.stack([W1[:, :_DH], W1[:, _DH:]])
    W2p = jnp.stack([W2f[:, :_DH], W2f[:, _DH:]])
    b1r = b1.reshape(1, 2 * _DH)
    b2r = jnp.pad(b2, (0, 2 * _DH - 40)).reshape(1, 2 * _DH)

    degp = _degrees(ones16, col)
    dinv, dinvf, u1 = _entry(degp, xp, W1c)
    dinv1 = dinvf.reshape(_NP)
    z1 = _layer(u1, dinv1, row, col)
    u2 = _mid(z1, dinv, b1r, W2p)
    z2 = _layer(u2, dinv1, row, col)
    out = _final(z2, dinv, b2r)
    return out[:_N]
